# baseline (speedup 1.0000x reference)
import functools

import jax
import jax.numpy as jnp
from jax import lax
from jax.experimental import pallas as pl
from jax.experimental.pallas import tpu as pltpu
from jax.experimental.pallas import tpu_sc as plsc

_B, _C, _D, _V = 1024, 20, 64, 100000
_NC, _NS = 2, 16
_NW = _NC * _NS
_ROWS = _B * _C
_RPW = _ROWS // _NW
_TV = 2048
_NT = (_V + _TV - 1) // _TV
_TVL = _V - (_NT - 1) * _TV
_NBUF = 4


def _sc_gather_body(table_hbm, idx_hbm, out_hbm, idx_v, rows_v, sem):
    wid = lax.axis_index("s") * _NC + lax.axis_index("c")
    base = wid * _RPW
    pltpu.sync_copy(idx_hbm.at[pl.ds(base, _RPW)], idx_v)
    pltpu.async_copy(table_hbm.at[idx_v], rows_v, sem).wait()
    pltpu.sync_copy(rows_v, out_hbm.at[pl.ds(base, _RPW)])


@functools.cache
def _get_sc_gather():
    return pl.kernel(
        _sc_gather_body,
        out_type=jax.ShapeDtypeStruct((_ROWS, _D), jnp.float32),
        mesh=plsc.VectorSubcoreMesh(core_axis_name="c", subcore_axis_name="s",
                                    num_cores=_NC, num_subcores=_NS),
        scratch_types=[
            pltpu.VMEM((_RPW,), jnp.int32),
            pltpu.VMEM((_RPW, _D), jnp.float32),
            pltpu.SemaphoreType.DMA,
        ],
        compiler_params=pltpu.CompilerParams(use_tc_tiling_on_sc=False),
    )


def _tc_body(emb_hbm, w_ref, b_hbm, out_hbm, x_ref, emb_v, b_v, obuf,
             obuf_last, sems, sem_last):
    i = pl.program_id(0)

    @pl.when(i == 0)
    def _():
        cp_e = pltpu.make_async_copy(emb_hbm, emb_v, sem_last)
        cp_b = pltpu.make_async_copy(b_hbm, b_v, sem_last)
        cp_e.start()
        cp_b.start()
        cp_e.wait()
        cp_b.wait()
        e = emb_v[...]
        ss = jnp.sum(e * e, axis=-1, keepdims=True)
        scale = jnp.minimum(1.0, 1.0 / jnp.maximum(jnp.sqrt(ss), 1e-7))
        x_ref[...] = jnp.mean(e * scale, axis=1)

    slot = lax.rem(i, _NBUF)

    @pl.when(i >= _NBUF)
    def _():
        j = i - _NBUF
        pltpu.make_async_copy(
            obuf.at[slot], out_hbm.at[:, pl.ds(j * _TV, _TV)], sems.at[slot]
        ).wait()

    y = lax.dot_general(
        x_ref[...].astype(jnp.bfloat16), w_ref[...].astype(jnp.bfloat16),
        (((1,), (1,)), ((), ())),
        preferred_element_type=jnp.float32) + b_v[pl.ds(i, 1), :]

    @pl.when(i < _NT - 1)
    def _():
        obuf[slot] = y
        pltpu.make_async_copy(
            obuf.at[slot], out_hbm.at[:, pl.ds(i * _TV, _TV)], sems.at[slot]
        ).start()

    @pl.when(i == _NT - 1)
    def _():
        obuf_last[...] = y[:, :_TVL]
        pltpu.make_async_copy(
            obuf_last, out_hbm.at[:, pl.ds((_NT - 1) * _TV, _TVL)], sem_last
        ).start()
        for j in range(_NT - _NBUF, _NT - 1):
            pltpu.make_async_copy(
                obuf.at[j % _NBUF], out_hbm.at[:, pl.ds(j * _TV, _TV)],
                sems.at[j % _NBUF]
            ).wait()
        pltpu.make_async_copy(
            obuf_last, out_hbm.at[:, pl.ds((_NT - 1) * _TV, _TVL)], sem_last
        ).wait()


_tc_project = pl.pallas_call(
    _tc_body,
    grid=(_NT,),
    in_specs=[
        pl.BlockSpec(memory_space=pl.ANY),
        pl.BlockSpec((_TV, _D), lambda i: (i, 0)),
        pl.BlockSpec(memory_space=pl.ANY),
    ],
    out_specs=pl.BlockSpec(memory_space=pl.ANY),
    out_shape=jax.ShapeDtypeStruct((_B, _V), jnp.float32),
    scratch_shapes=[
        pltpu.VMEM((_B, _D), jnp.float32),
        pltpu.VMEM((_B, _C, _D), jnp.float32),
        pltpu.VMEM((_NT, _TV), jnp.float32),
        pltpu.VMEM((_NBUF, _B, _TV), jnp.float32),
        pltpu.VMEM((_B, _TVL), jnp.float32),
        pltpu.SemaphoreType.DMA((_NBUF,)),
        pltpu.SemaphoreType.DMA,
    ],
    compiler_params=pltpu.CompilerParams(vmem_limit_bytes=100 * 1024 * 1024,
                                         needs_layout_passes=True),
)


def kernel(inputs_, emb_table, W, b):
    idx = inputs_.reshape(-1).astype(jnp.int32)
    rows = _get_sc_gather()(emb_table, idx)
    emb = rows.reshape(_B, _C, _D)
    bp = jnp.pad(b, (0, _NT * _TV - _V)).reshape(_NT, _TV)
    return _tc_project(emb, W, bp)


# speedup vs baseline: 2.1538x; 2.1538x over previous
import functools

import jax
import jax.numpy as jnp
from jax import lax
from jax.experimental import pallas as pl
from jax.experimental.pallas import tpu as pltpu
from jax.experimental.pallas import tpu_sc as plsc

_B, _C, _D, _V = 1024, 20, 64, 100000
_NC, _NS = 2, 16
_NW = _NC * _NS
_ROWS = _B * _C
_RPW = _ROWS // _NW
_TV = 2048
_NT = (_V + _TV - 1) // _TV
_TVL = _V - (_NT - 1) * _TV
_NBUF = 4


def _sc_gather_body(table_hbm, idx_hbm, out_hbm, idx_v, rows_v, sem):
    wid = lax.axis_index("s") * _NC + lax.axis_index("c")
    base = wid * _RPW
    pltpu.sync_copy(idx_hbm.at[pl.ds(base, _RPW)], idx_v)
    pltpu.async_copy(table_hbm.at[idx_v], rows_v, sem).wait()
    pltpu.sync_copy(rows_v, out_hbm.at[pl.ds(base, _RPW)])


@functools.cache
def _get_sc_gather():
    return pl.kernel(
        _sc_gather_body,
        out_type=jax.ShapeDtypeStruct((_ROWS, _D), jnp.float32),
        mesh=plsc.VectorSubcoreMesh(core_axis_name="c", subcore_axis_name="s",
                                    num_cores=_NC, num_subcores=_NS),
        scratch_types=[
            pltpu.VMEM((_RPW,), jnp.int32),
            pltpu.VMEM((_RPW, _D), jnp.float32),
            pltpu.SemaphoreType.DMA,
        ],
        compiler_params=pltpu.CompilerParams(use_tc_tiling_on_sc=False),
    )


def _tc_body(emb_hbm, wt_ref, b_ref, out_hbm, xt_ref, emb_v, obuf, sems,
             sem_ld):
    i = pl.program_id(0)

    @pl.when(i == 0)
    def _():
        cp = pltpu.make_async_copy(emb_hbm, emb_v, sem_ld)
        cp.start()
        cp.wait()
        e = emb_v[...]
        ss = jnp.sum(e * e, axis=-1, keepdims=True)
        scale = jnp.minimum(1.0, 1.0 / jnp.maximum(jnp.sqrt(ss), 1e-7))
        x = jnp.mean(e * scale, axis=1)
        xt_ref[...] = jnp.swapaxes(x, 0, 1)

    slot = lax.rem(i, _NBUF)

    @pl.when(i >= _NBUF)
    def _():
        j = i - _NBUF
        pltpu.make_async_copy(
            obuf.at[slot], out_hbm.at[pl.ds(j * _TV, _TV), :], sems.at[slot]
        ).wait()

    obuf[slot] = lax.dot_general(
        wt_ref[...].astype(jnp.bfloat16), xt_ref[...].astype(jnp.bfloat16),
        (((0,), (0,)), ((), ())),
        preferred_element_type=jnp.float32) + b_ref[...]

    @pl.when(i < _NT - 1)
    def _():
        pltpu.make_async_copy(
            obuf.at[slot], out_hbm.at[pl.ds(i * _TV, _TV), :], sems.at[slot]
        ).start()

    @pl.when(i == _NT - 1)
    def _():
        pltpu.make_async_copy(
            obuf.at[slot, :_TVL, :],
            out_hbm.at[pl.ds((_NT - 1) * _TV, _TVL), :], sems.at[slot]
        ).start()
        for j in range(_NT - _NBUF, _NT - 1):
            pltpu.make_async_copy(
                obuf.at[j % _NBUF], out_hbm.at[pl.ds(j * _TV, _TV), :],
                sems.at[j % _NBUF]
            ).wait()
        pltpu.make_async_copy(
            obuf.at[(_NT - 1) % _NBUF, :_TVL, :],
            out_hbm.at[pl.ds((_NT - 1) * _TV, _TVL), :],
            sems.at[(_NT - 1) % _NBUF]
        ).wait()


_tc_project = pl.pallas_call(
    _tc_body,
    grid=(_NT,),
    in_specs=[
        pl.BlockSpec(memory_space=pl.ANY),
        pl.BlockSpec((_D, _TV), lambda i: (0, i)),
        pl.BlockSpec((_TV, 1), lambda i: (i, 0)),
    ],
    out_specs=pl.BlockSpec(memory_space=pl.ANY),
    out_shape=jax.ShapeDtypeStruct((_V, _B), jnp.float32),
    scratch_shapes=[
        pltpu.VMEM((_D, _B), jnp.float32),
        pltpu.VMEM((_B, _C, _D), jnp.float32),
        pltpu.VMEM((_NBUF, _TV, _B), jnp.float32),
        pltpu.SemaphoreType.DMA((_NBUF,)),
        pltpu.SemaphoreType.DMA,
    ],
    compiler_params=pltpu.CompilerParams(vmem_limit_bytes=100 * 1024 * 1024),
)


def kernel(inputs_, emb_table, W, b):
    idx = inputs_.reshape(-1).astype(jnp.int32)
    rows = _get_sc_gather()(emb_table, idx)
    emb = rows.reshape(_B, _C, _D)
    logits_t = _tc_project(emb, W.T, b.reshape(_V, 1))
    return logits_t.T


# speedup vs baseline: 2.6483x; 1.2296x over previous
import functools

import jax
import jax.numpy as jnp
from jax import lax
from jax.experimental import pallas as pl
from jax.experimental.pallas import tpu as pltpu
from jax.experimental.pallas import tpu_sc as plsc

_B, _C, _D, _V = 1024, 20, 64, 100000
_NC, _NS = 2, 16
_NW = _NC * _NS
_ROWS = _B * _C
_RPW = _ROWS // _NW
_TV = 2048
_NT = (_V + _TV - 1) // _TV
_TVL = _V - (_NT - 1) * _TV
_NBUF = 4


def _sc_gather_body(table_hbm, idx_hbm, out_hbm, idx_v, rows_v, sem):
    wid = lax.axis_index("s") * _NC + lax.axis_index("c")
    base = wid * _RPW
    pltpu.sync_copy(idx_hbm.at[pl.ds(base, _RPW)], idx_v)
    pltpu.async_copy(table_hbm.at[idx_v], rows_v, sem).wait()
    pltpu.sync_copy(rows_v, out_hbm.at[pl.ds(base, _RPW)])


@functools.cache
def _get_sc_gather():
    return pl.kernel(
        _sc_gather_body,
        out_type=jax.ShapeDtypeStruct((_ROWS, _D), jnp.float32),
        mesh=plsc.VectorSubcoreMesh(core_axis_name="c", subcore_axis_name="s",
                                    num_cores=_NC, num_subcores=_NS),
        scratch_types=[
            pltpu.VMEM((_RPW,), jnp.int32),
            pltpu.VMEM((_RPW, _D), jnp.float32),
            pltpu.SemaphoreType.DMA,
        ],
        compiler_params=pltpu.CompilerParams(use_tc_tiling_on_sc=False),
    )


def _tc_body(emb_hbm, wt_ref, b_hbm, out_hbm, xt_ref, emb_v, b_v,
             obuf, sems, sem_ld):
    i = pl.program_id(0)

    @pl.when(i == 0)
    def _():
        cp_e = pltpu.make_async_copy(emb_hbm, emb_v, sem_ld)
        cp_b = pltpu.make_async_copy(b_hbm, b_v, sem_ld)
        cp_e.start()
        cp_b.start()
        cp_e.wait()
        cp_b.wait()
        e = emb_v[...]
        ss = jnp.sum(e * e, axis=-1, keepdims=True)
        scale = jnp.minimum(1.0, 1.0 / jnp.maximum(jnp.sqrt(ss), 1e-7))
        x = jnp.mean(e * scale, axis=1)
        xt_ref[...] = jnp.swapaxes(x, 0, 1)

    slot = lax.rem(i, _NBUF)

    @pl.when(i >= _NBUF)
    def _():
        j = i - _NBUF
        pltpu.make_async_copy(
            obuf.at[slot], out_hbm.at[pl.ds(j * _TV, _TV), :], sems.at[slot]
        ).wait()

    onehot = (lax.broadcasted_iota(jnp.int32, (_NT, 1), 0) == i)
    bcol = lax.dot_general(
        b_v[...], onehot.astype(jnp.float32), (((0,), (0,)), ((), ())),
        preferred_element_type=jnp.float32)
    obuf[slot] = lax.dot_general(
        wt_ref[...].astype(jnp.bfloat16), xt_ref[...].astype(jnp.bfloat16),
        (((0,), (0,)), ((), ())),
        preferred_element_type=jnp.float32) + bcol

    @pl.when(i < _NT - 1)
    def _():
        pltpu.make_async_copy(
            obuf.at[slot], out_hbm.at[pl.ds(i * _TV, _TV), :], sems.at[slot]
        ).start()

    @pl.when(i == _NT - 1)
    def _():
        pltpu.make_async_copy(
            obuf.at[slot, :_TVL, :],
            out_hbm.at[pl.ds((_NT - 1) * _TV, _TVL), :], sems.at[slot]
        ).start()
        for j in range(_NT - _NBUF, _NT - 1):
            pltpu.make_async_copy(
                obuf.at[j % _NBUF], out_hbm.at[pl.ds(j * _TV, _TV), :],
                sems.at[j % _NBUF]
            ).wait()
        pltpu.make_async_copy(
            obuf.at[(_NT - 1) % _NBUF, :_TVL, :],
            out_hbm.at[pl.ds((_NT - 1) * _TV, _TVL), :],
            sems.at[(_NT - 1) % _NBUF]
        ).wait()


_tc_project = pl.pallas_call(
    _tc_body,
    grid=(_NT,),
    in_specs=[
        pl.BlockSpec(memory_space=pl.ANY),
        pl.BlockSpec((_D, _TV), lambda i: (0, i)),
        pl.BlockSpec(memory_space=pl.ANY),
    ],
    out_specs=pl.BlockSpec(memory_space=pl.ANY),
    out_shape=jax.ShapeDtypeStruct((_V, _B), jnp.float32),
    scratch_shapes=[
        pltpu.VMEM((_D, _B), jnp.float32),
        pltpu.VMEM((_B, _C, _D), jnp.float32),
        pltpu.VMEM((_NT, _TV), jnp.float32),
        pltpu.VMEM((_NBUF, _TV, _B), jnp.float32),
        pltpu.SemaphoreType.DMA((_NBUF,)),
        pltpu.SemaphoreType.DMA,
    ],
    compiler_params=pltpu.CompilerParams(vmem_limit_bytes=100 * 1024 * 1024),
)


def kernel(inputs_, emb_table, W, b):
    idx = inputs_.reshape(-1).astype(jnp.int32)
    rows = _get_sc_gather()(emb_table, idx)
    emb = rows.reshape(_B, _C, _D)
    bp = jnp.pad(b, (0, _NT * _TV - _V)).reshape(_NT, _TV)
    logits_t = _tc_project(emb, W.T, bp)
    return logits_t.T
